# baseline (speedup 1.0000x reference)
import jax
import jax.numpy as jnp
from jax import lax
from jax.experimental import pallas as pl
from jax.experimental.pallas import tpu as pltpu, tpu_sc as plsc

NUM_USERS = 1000000
DIM = 32
K = 20
BATCH = 16384

NC = 2
NS = 16
NW = NC * NS
BPW = BATCH // NW
CH = 64
NCHUNK = BPW // CH
NIDXROW = CH * K // 128
NR = BPW // 128

_GDN = lax.GatherDimensionNumbers(
    offset_dims=(), collapsed_slice_dims=(0,), start_index_map=(0,))


def _vgather(vec, ixvec):
    return lax.gather(vec, ixvec[:, None], _GDN, (1,),
                      mode=lax.GatherScatterMode.PROMISE_IN_BOUNDS)


def _body(uid_hbm, w_hbm, nidx_hbm, nw_hbm, upref_hbm, usoc_hbm,
          uid_v, upref_v, ria_v, rib_v, nia_v, nib_v, nwa_v, nwb_v,
          cidx_v, nemb_v, usoc_v, sem_a, sem_b):
    wid = lax.axis_index("s") * NC + lax.axis_index("c")
    base = wid * BPW
    lanes = lax.iota(jnp.int32, 16)

    pltpu.sync_copy(uid_hbm.at[pl.ds(base, BPW)], uid_v)
    cp_pref = pltpu.async_copy(w_hbm.at[uid_v], upref_v, sem_a)

    @pl.loop(0, BPW // 16)
    def _mk(m):
        u16 = uid_v[pl.ds(m * 16, 16)]
        ra = (u16 * 5) >> 2
        ria_v[m >> 3, pl.ds((m & 7) * 16, 16)] = ra
        rib_v[m >> 3, pl.ds((m & 7) * 16, 16)] = ra + 1

    tbl = []
    for j in range(NR):
        s = pl.ds(j * 128, 128)
        tbl.append(pltpu.async_copy(
            nidx_hbm.at[ria_v.at[j]], nia_v.at[s], sem_b))
        tbl.append(pltpu.async_copy(
            nidx_hbm.at[rib_v.at[j]], nib_v.at[s], sem_b))
        tbl.append(pltpu.async_copy(
            nw_hbm.at[ria_v.at[j]], nwa_v.at[s], sem_b))
        tbl.append(pltpu.async_copy(
            nw_hbm.at[rib_v.at[j]], nwb_v.at[s], sem_b))

    cp_pref.wait()
    pltpu.sync_copy(upref_v, upref_hbm.at[pl.ds(base, BPW)])
    for cp in tbl:
        cp.wait()

    def _rot(u):
        grp = (u >> 4) << 4
        u16 = uid_v[pl.ds(grp, 16)]
        ubc = _vgather(u16, jnp.zeros((16,), jnp.int32) + (u & 15))
        return (ubc * 4) & 15

    def _window(rowa, rowb, s):
        ix = (lanes + s) & 15
        return jnp.where(lanes < 16 - s, _vgather(rowa, ix),
                         _vgather(rowb, ix))

    for g in range(NCHUNK):
        @pl.loop(0, CH)
        def _expand(c):
            u = g * CH + c
            s0 = _rot(u)
            v_lo = _window(nia_v[u, pl.ds(0, 16)], nib_v[u, pl.ds(0, 16)],
                           s0)
            v_hi = _window(nia_v[u, pl.ds(0, 16)], nib_v[u, pl.ds(0, 16)],
                           s0 + 4)
            p_lo = c * K + lanes
            p_hi = c * K + 4 + lanes
            plsc.store_scatter(cidx_v, [p_lo >> 7, p_lo & 127], v_lo)
            plsc.store_scatter(cidx_v, [p_hi >> 7, p_hi & 127], v_hi,
                               mask=lanes >= 12)

        copies = [
            pltpu.async_copy(
                w_hbm.at[cidx_v.at[j]],
                nemb_v.at[pl.ds(j * 128, 128)], sem_a)
            for j in range(NIDXROW)
        ]
        for cp in copies:
            cp.wait()

        @pl.loop(0, CH)
        def _compute(c):
            u = g * CH + c
            s0 = _rot(u)
            w_lo = _window(nwa_v[u, pl.ds(0, 16)], nwb_v[u, pl.ds(0, 16)],
                           s0)
            w_hi = _window(nwa_v[u, pl.ds(0, 16)], nwb_v[u, pl.ds(0, 16)],
                           s0 + 4)
            acc0 = jnp.zeros((16,), jnp.float32)
            acc1 = jnp.zeros((16,), jnp.float32)
            for k in range(K):
                lane = k if k < 16 else k - 4
                src = w_lo if k < 16 else w_hi
                wk = _vgather(src, jnp.full((16,), lane, jnp.int32))
                acc0 = acc0 + wk * nemb_v[c * K + k, pl.ds(0, 16)]
                acc1 = acc1 + wk * nemb_v[c * K + k, pl.ds(16, 16)]
            usoc_v[c, pl.ds(0, 16)] = acc0
            usoc_v[c, pl.ds(16, 16)] = acc1

        pltpu.sync_copy(usoc_v, usoc_hbm.at[pl.ds(base + g * CH, CH)])


@jax.jit
def _run(user_ids, W, neighbor_idx, neighbor_w):
    nidx16 = neighbor_idx.reshape(NUM_USERS * K // 16, 16)
    nw16 = neighbor_w.reshape(NUM_USERS * K // 16, 16)
    mesh = plsc.VectorSubcoreMesh(core_axis_name="c", subcore_axis_name="s")
    f = pl.kernel(
        _body,
        out_type=(
            jax.ShapeDtypeStruct((BATCH, DIM), jnp.float32),
            jax.ShapeDtypeStruct((BATCH, DIM), jnp.float32),
        ),
        mesh=mesh,
        compiler_params=pltpu.CompilerParams(
            needs_layout_passes=False, use_tc_tiling_on_sc=False),
        scratch_types=[
            pltpu.VMEM((BPW,), jnp.int32),
            pltpu.VMEM((BPW, DIM), jnp.float32),
            pltpu.VMEM((NR, 128), jnp.int32),
            pltpu.VMEM((NR, 128), jnp.int32),
            pltpu.VMEM((BPW, 16), jnp.int32),
            pltpu.VMEM((BPW, 16), jnp.int32),
            pltpu.VMEM((BPW, 16), jnp.float32),
            pltpu.VMEM((BPW, 16), jnp.float32),
            pltpu.VMEM((NIDXROW, 128), jnp.int32),
            pltpu.VMEM((CH * K, DIM), jnp.float32),
            pltpu.VMEM((CH, DIM), jnp.float32),
            pltpu.SemaphoreType.DMA,
            pltpu.SemaphoreType.DMA,
        ],
    )
    return f(user_ids, W, nidx16, nw16)


def kernel(user_ids, W, neighbor_idx, neighbor_w):
    return _run(user_ids, W, neighbor_idx, neighbor_w)
